# baseline (speedup 1.0000x reference)
import functools

import jax
import jax.numpy as jnp
from jax import lax
from jax.experimental import pallas as pl
from jax.experimental.pallas import tpu as pltpu
from jax.experimental.pallas import tpu_sc as plsc

TEMP = 0.1
NC = 2
NS = 16
LN = 16

BLK = 2048


@functools.lru_cache(maxsize=None)
def _make_deg_kernel(n_acc, e_pad):
    e_pt = e_pad // (NC * NS)
    nblk = e_pt // 1024
    assert nblk * 1024 == e_pt
    mesh = plsc.VectorSubcoreMesh(core_axis_name="c", subcore_axis_name="s")

    @functools.partial(
        pl.kernel, mesh=mesh,
        compiler_params=pltpu.CompilerParams(use_tc_tiling_on_sc=False, needs_layout_passes=False),
        out_type=(jax.ShapeDtypeStruct((NC * NS * n_acc,), jnp.float32),
                  jax.ShapeDtypeStruct((NC * NS * n_acc,), jnp.float32)),
        scratch_types=[
            pltpu.VMEM((1024,), jnp.int32),
            pltpu.VMEM((1024,), jnp.int32),
            pltpu.VMEM((n_acc,), jnp.float32),
            pltpu.SemaphoreType.DMA,
            pltpu.SemaphoreType.DMA,
        ],
    )
    def k(ua_hbm, ub_hbm, outa_hbm, outb_hbm, u0, u1, hist, sem0, sem1):
        c = lax.axis_index("c")
        s = lax.axis_index("s")
        w = c * NS + s
        ones = jnp.ones((LN,), jnp.float32)
        bufs = ((u0, sem0), (u1, sem1))

        for u_hbm, out_hbm in ((ua_hbm, outa_hbm), (ub_hbm, outb_hbm)):
            def zh(i, _):
                hist[pl.ds(i * LN, LN)] = jnp.zeros((LN,), jnp.float32)
                return 0
            lax.fori_loop(0, n_acc // LN, zh, 0)

            def start_load(buf, b):
                u_vm, sem = buf
                return pltpu.async_copy(
                    u_hbm.at[pl.ds(w * e_pt + b * 1024, 1024)], u_vm, sem)

            def histogram(buf):
                u_vm, sem = buf
                pltpu.make_async_copy(
                    u_hbm.at[pl.ds(0, 1024)], u_vm, sem).wait()
                for j in range(1024 // LN):
                    iv = u_vm[pl.ds(j * LN, LN)]
                    plsc.addupdate_scatter(hist, [iv], ones)

            start_load(bufs[0], 0)
            last = nblk - 1 if nblk % 2 else nblk - 2

            def blk(bb, _):
                b0 = 2 * bb
                start_load(bufs[1], b0 + 1)
                histogram(bufs[0])
                start_load(bufs[0], jnp.minimum(b0 + 2, last))
                histogram(bufs[1])
                return 0
            lax.fori_loop(0, nblk // 2, blk, 0)
            if nblk % 2:
                histogram(bufs[0])
            else:
                pltpu.make_async_copy(u_hbm.at[pl.ds(0, 1024)], u0, sem0).wait()

            pltpu.sync_copy(hist, out_hbm.at[pl.ds(w * n_acc, n_acc)])

    return k


@functools.lru_cache(maxsize=None)
def _make_scatter_kernel(n_acc, e_pad):
    e_pc = e_pad // NS
    nblk = e_pc // BLK
    assert nblk * BLK == e_pc
    RPB = BLK // 128
    rows_pc = e_pc // 128
    slc = n_acc // NS
    mesh = plsc.VectorSubcoreMesh(core_axis_name="c", subcore_axis_name="s")

    @functools.partial(
        pl.kernel, mesh=mesh,
        compiler_params=pltpu.CompilerParams(use_tc_tiling_on_sc=False, needs_layout_passes=False),
        out_type=jax.ShapeDtypeStruct((4 * n_acc, LN), jnp.float32),
        scratch_types=[
            pltpu.VMEM((RPB, 128), jnp.int32),
            pltpu.VMEM((RPB, 128), jnp.int32),
            pltpu.VMEM((RPB, 128), jnp.int32),
            pltpu.VMEM((BLK, LN), jnp.float32),
            pltpu.VMEM_SHARED((n_acc, LN), jnp.float32),
            pltpu.SemaphoreType.DMA,
            pltpu.SemaphoreType.DMA,
        ],
    )
    def k(y_hbm, u_hbm, v_hbm, out_hbm, u_vm, uo_vm, v_vm, rows_vm, acc, sg, ss):
        c = lax.axis_index("c")
        s = lax.axis_index("s")

        for qi in range(2):
            q = 2 * c + qi
            qoff = q * n_acc

            def zrows(i, _):
                rows_vm[i, :] = jnp.zeros((LN,), jnp.float32)
                return 0
            lax.fori_loop(0, BLK, zrows, 0)
            nz_full = slc // BLK
            for zi in range(nz_full):
                pltpu.sync_copy(rows_vm, acc.at[pl.ds(s * slc + zi * BLK, BLK)])
            rem = slc - nz_full * BLK
            if rem:
                pltpu.sync_copy(rows_vm.at[pl.ds(0, rem)],
                                acc.at[pl.ds(s * slc + nz_full * BLK, rem)])
            plsc.subcore_barrier()

            def blk_body(b, _):
                row0 = s * rows_pc + b * RPB
                pltpu.sync_copy(u_hbm.at[pl.ds(row0, RPB)], u_vm)
                pltpu.sync_copy(v_hbm.at[pl.ds(row0, RPB)], v_vm)
                for j in range(RPB):
                    for l in range(128 // LN):
                        uo_vm[j, pl.ds(l * LN, LN)] = (
                            u_vm[j, pl.ds(l * LN, LN)] + qoff)
                gathers = [
                    pltpu.async_copy(y_hbm.at[uo_vm.at[j]],
                                     rows_vm.at[pl.ds(j * 128, 128)], sg)
                    for j in range(RPB)
                ]
                for g in gathers:
                    g.wait()
                scatters = [
                    pltpu.async_copy(rows_vm.at[pl.ds(j * 128, 128)],
                                     acc.at[v_vm.at[j]], ss, add=True)
                    for j in range(RPB)
                ]
                for sc in scatters:
                    sc.wait()
                return 0
            lax.fori_loop(0, nblk, blk_body, 0)

            plsc.subcore_barrier()
            pltpu.sync_copy(acc.at[pl.ds(s * slc, slc)],
                            out_hbm.at[pl.ds(qoff + s * slc, slc)])
            plsc.subcore_barrier()

    return k


def _quarter(x_pad):
    n_acc = x_pad.shape[0]
    return jnp.transpose(x_pad.reshape(n_acc, 4, LN), (1, 0, 2)).reshape(4 * n_acc, LN)


def _unquarter_rows(tab, idx):
    n_acc = tab.shape[0] // 4
    q = tab.reshape(4, n_acc, LN)[:, idx, :]
    return jnp.transpose(q, (1, 0, 2)).reshape(idx.shape[0], 64)


def _edges(emb_a, edge_index, n):
    na = emb_a.shape[0]
    e = edge_index.shape[1]
    src = edge_index[0]
    dst = edge_index[1] + na
    u = jnp.concatenate([src, dst])
    v = jnp.concatenate([dst, src])
    blk2 = NS * 2 * BLK
    e_pad = ((2 * e + blk2 - 1) // blk2) * blk2
    pad = e_pad - 2 * e
    u = jnp.concatenate([u, jnp.full((pad,), n, jnp.int32)])
    v = jnp.concatenate([v, jnp.full((pad,), n, jnp.int32)])
    return u, v, e_pad


def _propagate_pair(g1, g2, n_acc):
    n = g1[0].shape[0] + g1[1].shape[0]
    u1, v1, e_pad = _edges(g1[0], g1[2], n)
    u2, v2, _ = _edges(g2[0], g2[2], n)

    d1p, d2p = _make_deg_kernel(n_acc, e_pad)(u1, u2)
    scat = _make_scatter_kernel(n_acc, e_pad)

    outs = []
    for (emb_a, emb_b, _), u, v, dp in ((g1, u1, v1, d1p), (g2, u2, v2, d2p)):
        deg = dp.reshape(NC * NS, n_acc).sum(axis=0)
        rs = lax.rsqrt(jnp.maximum(deg, 1.0))[:, None]
        x0 = jnp.zeros((n_acc, 64), jnp.float32).at[:n].set(
            jnp.concatenate([emb_a, emb_b], axis=0))
        rs4 = jnp.tile(rs, (4, 1))
        u2d = u.reshape(e_pad // 128, 128)
        v2d = v.reshape(e_pad // 128, 128)
        y0 = _quarter(x0 * rs)
        t1 = scat(y0, u2d, v2d)
        t2 = scat(t1 * (rs4 * rs4), u2d, v2d)
        outs.append((x0, t1, t2, rs))
    return outs


def _final_rows(x0, t1, t2, rs, idx):
    r = rs[idx]
    return (x0[idx] + r * (_unquarter_rows(t1, idx) + _unquarter_rows(t2, idx))) * (1.0 / 3.0)


def _fuse_rows(e1, e2, Wg, bg):
    h = jnp.concatenate([e1, e2], axis=-1)
    out = jnp.zeros_like(e1)
    for g in range(Wg.shape[0]):
        gate = jax.nn.sigmoid(jnp.dot(h, Wg[g], preferred_element_type=jnp.float32) + bg[g])
        out = out + gate * e1 + (1.0 - gate) * e2
    return out * (1.0 / Wg.shape[0])


def _normalize(z):
    return z / (jnp.sqrt(jnp.sum(z * z, axis=-1, keepdims=True)) + 1e-8)


def _contrast_part(z1_blk, z2_full, base):
    nb = z1_blk.shape[0]
    n = z2_full.shape[0]
    sim = jnp.dot(_normalize(z1_blk), _normalize(z2_full).T,
                  preferred_element_type=jnp.float32) * (1.0 / TEMP)
    m = jnp.max(sim, axis=-1, keepdims=True)
    lse = jnp.log(jnp.sum(jnp.exp(sim - m), axis=-1, keepdims=True)) + m
    rows = jax.lax.broadcasted_iota(jnp.int32, (nb, n), 0)
    cols = jax.lax.broadcasted_iota(jnp.int32, (nb, n), 1)
    diag = jnp.sum(jnp.where(cols == rows + base, sim, 0.0), axis=-1, keepdims=True)
    return jnp.sum(diag - lse)


def _cpt_fuse_kernel_body(c_ref, w_ref, Wg_ref, bg_ref, out_ref):
    out_ref[...] = _fuse_rows(c_ref[...], w_ref[...], Wg_ref[...], bg_ref[...])


def _cpt_fuse(cpt_c, cpt_w, Wg, bg):
    npad = cpt_c.shape[0]
    return pl.pallas_call(
        _cpt_fuse_kernel_body,
        out_shape=jax.ShapeDtypeStruct((npad, 64), jnp.float32),
    )(cpt_c, cpt_w, Wg, bg)


def _head_kernel_body(s_cse, s_wse, s_csc, s_wsc, e_c, e_w, e_w_full,
                      z1c, z2c_full, cpt_mean,
                      Wg_se, bg_se, Wg_sc, bg_sc, Wg_stu, bg_stu, Wg_exer, bg_exer,
                      W1, b1, W2, b2, Wk1, bk1, Wk2, bk2,
                      pred_ref, ks_ref, ce_ref, cc_ref):
    i = pl.program_id(0)
    nb = s_cse.shape[0]
    stu_se = _fuse_rows(s_cse[...], s_wse[...], Wg_se[...], bg_se[...])
    stu_sc = _fuse_rows(s_csc[...], s_wsc[...], Wg_sc[...], bg_sc[...])
    b_stu = _fuse_rows(stu_se, stu_sc, Wg_stu[...], bg_stu[...])
    b_exer = _fuse_rows(e_c[...], e_w[...], Wg_exer[...], bg_exer[...])

    h = jnp.concatenate([b_stu, b_exer, cpt_mean[...]], axis=-1)
    h = jax.nn.relu(jnp.dot(h, W1[...], preferred_element_type=jnp.float32) + b1[...])
    pred_ref[...] = jax.nn.sigmoid(
        jnp.dot(h, W2[...], preferred_element_type=jnp.float32) + b2[...])
    kh = jax.nn.relu(jnp.dot(b_stu, Wk1[...], preferred_element_type=jnp.float32) + bk1[...])
    ks_ref[...] = jax.nn.sigmoid(
        jnp.dot(kh, Wk2[...], preferred_element_type=jnp.float32) + bk2[...])

    base = i * nb
    ce_ref[...] = _contrast_part(e_c[...], e_w_full[...], base).reshape(1, 1, 1)
    cc_ref[...] = _contrast_part(z1c[...], z2c_full[...], base).reshape(1, 1, 1)


def _heads(s_cse, s_wse, s_csc, s_wsc, e_c, e_w, z1c, z2c, cpt_mean,
           Wg_se, bg_se, Wg_sc, bg_sc, Wg_stu, bg_stu, Wg_exer, bg_exer,
           W1, b1, W2, b2, Wk1, bk1, Wk2, bk2):
    B = s_cse.shape[0]
    NB = 512
    G = B // NB
    C_out = Wk2.shape[1]
    b1, b2, bk1, bk2 = b1[None, :], b2[None, :], bk1[None, :], bk2[None, :]

    def rows(i):
        return (i, 0)

    def whole(i):
        return (0, 0)

    row_spec = pl.BlockSpec((NB, 64), rows)
    full_spec = pl.BlockSpec((B, 64), whole)
    weights = (Wg_se, bg_se, Wg_sc, bg_sc, Wg_stu, bg_stu, Wg_exer, bg_exer,
               W1, b1, W2, b2, Wk1, bk1, Wk2, bk2)

    pred, ks, ce, cc = pl.pallas_call(
        _head_kernel_body,
        grid=(G,),
        in_specs=[row_spec] * 6
        + [full_spec]
        + [row_spec, full_spec, row_spec]
        + [pl.BlockSpec(w.shape, lambda i, nd=w.ndim: (0,) * nd)
           for w in weights],
        out_specs=[
            pl.BlockSpec((NB, 1), rows),
            pl.BlockSpec((NB, C_out), rows),
            pl.BlockSpec((1, 1, 1), lambda i: (i, 0, 0)),
            pl.BlockSpec((1, 1, 1), lambda i: (i, 0, 0)),
        ],
        out_shape=[
            jax.ShapeDtypeStruct((B, 1), jnp.float32),
            jax.ShapeDtypeStruct((B, C_out), jnp.float32),
            jax.ShapeDtypeStruct((G, 1, 1), jnp.float32),
            jax.ShapeDtypeStruct((G, 1, 1), jnp.float32),
        ],
    )(s_cse, s_wse, s_csc, s_wsc, e_c, e_w, e_w, z1c, z2c, cpt_mean, *weights)
    c_exer = -jnp.sum(ce) / B
    c_cpt = -jnp.sum(cc) / B
    return pred[:, 0], ks, c_exer, c_cpt


def kernel(stu_ids, exer_ids, cpt_ids, labels, adj_correct_se, adj_wrong_se,
           adj_correct_sc, adj_wrong_sc,
           emb_stu_cse, emb_exer_c, emb_stu_wse, emb_exer_w,
           emb_stu_csc, emb_cpt_c, emb_stu_wsc, emb_cpt_w,
           Wg_se, bg_se, Wg_sc, bg_sc, Wg_stu, bg_stu, Wg_exer, bg_exer,
           Wg_cpt, bg_cpt, W1, b1, W2, b2, Wk1, bk1, Wk2, bk2):
    S = emb_stu_cse.shape[0]
    C = emb_cpt_c.shape[0]
    N_SE = 70144
    N_SC = 51200

    p_cse, p_wse = _propagate_pair(
        (emb_stu_cse, emb_exer_c, adj_correct_se),
        (emb_stu_wse, emb_exer_w, adj_wrong_se), N_SE)
    p_csc, p_wsc = _propagate_pair(
        (emb_stu_csc, emb_cpt_c, adj_correct_sc),
        (emb_stu_wsc, emb_cpt_w, adj_wrong_sc), N_SC)

    exer_nodes = S + exer_ids
    cpt_nodes = S + jnp.arange(C, dtype=jnp.int32)

    stu_c_se_b = _final_rows(*p_cse, stu_ids)
    stu_w_se_b = _final_rows(*p_wse, stu_ids)
    stu_c_sc_b = _final_rows(*p_csc, stu_ids)
    stu_w_sc_b = _final_rows(*p_wsc, stu_ids)
    exer_c_b = _final_rows(*p_cse, exer_nodes)
    exer_w_b = _final_rows(*p_wse, exer_nodes)
    cpt_c_t = _final_rows(*p_csc, cpt_nodes)
    cpt_w_t = _final_rows(*p_wsc, cpt_nodes)

    pad_c = jnp.zeros((1024 - C, 64), jnp.float32)
    cpt_final = _cpt_fuse(jnp.concatenate([cpt_c_t, pad_c]),
                          jnp.concatenate([cpt_w_t, pad_c]), Wg_cpt, bg_cpt)[:C]
    cpt_mean = jnp.mean(cpt_final[cpt_ids], axis=1)

    cpt_batch = cpt_ids[:, 0]
    predictions, knowledge_state, c_exer, c_cpt = _heads(
        stu_c_se_b, stu_w_se_b, stu_c_sc_b, stu_w_sc_b, exer_c_b, exer_w_b,
        cpt_c_t[cpt_batch], cpt_w_t[cpt_batch], cpt_mean,
        Wg_se, bg_se, Wg_sc, bg_sc, Wg_stu, bg_stu, Wg_exer, bg_exer,
        W1, b1, W2, b2, Wk1, bk1, Wk2, bk2)
    return (predictions, knowledge_state, c_exer, c_cpt)


# speedup vs baseline: 1.3152x; 1.3152x over previous
import functools

import jax
import jax.numpy as jnp
from jax import lax
from jax.experimental import pallas as pl
from jax.experimental.pallas import tpu as pltpu
from jax.experimental.pallas import tpu_sc as plsc

TEMP = 0.1
NC = 2
NS = 16
LN = 16

BLK = 1024


@functools.lru_cache(maxsize=None)
def _make_deg_kernel(n_acc, e_pad):
    e_pt = e_pad // (NC * NS)
    nblk = e_pt // 1024
    assert nblk * 1024 == e_pt
    mesh = plsc.VectorSubcoreMesh(core_axis_name="c", subcore_axis_name="s")

    @functools.partial(
        pl.kernel, mesh=mesh,
        compiler_params=pltpu.CompilerParams(use_tc_tiling_on_sc=False, needs_layout_passes=False),
        out_type=(jax.ShapeDtypeStruct((NC * NS * n_acc,), jnp.float32),
                  jax.ShapeDtypeStruct((NC * NS * n_acc,), jnp.float32)),
        scratch_types=[
            pltpu.VMEM((1024,), jnp.int32),
            pltpu.VMEM((1024,), jnp.int32),
            pltpu.VMEM((n_acc,), jnp.float32),
            pltpu.SemaphoreType.DMA,
            pltpu.SemaphoreType.DMA,
        ],
    )
    def k(ua_hbm, ub_hbm, outa_hbm, outb_hbm, u0, u1, hist, sem0, sem1):
        c = lax.axis_index("c")
        s = lax.axis_index("s")
        w = c * NS + s
        ones = jnp.ones((LN,), jnp.float32)
        bufs = ((u0, sem0), (u1, sem1))

        for u_hbm, out_hbm in ((ua_hbm, outa_hbm), (ub_hbm, outb_hbm)):
            def zh(i, _):
                hist[pl.ds(i * LN, LN)] = jnp.zeros((LN,), jnp.float32)
                return 0
            lax.fori_loop(0, n_acc // LN, zh, 0)

            def start_load(buf, b):
                u_vm, sem = buf
                return pltpu.async_copy(
                    u_hbm.at[pl.ds(w * e_pt + b * 1024, 1024)], u_vm, sem)

            def histogram(buf):
                u_vm, sem = buf
                pltpu.make_async_copy(
                    u_hbm.at[pl.ds(0, 1024)], u_vm, sem).wait()
                for j in range(1024 // LN):
                    iv = u_vm[pl.ds(j * LN, LN)]
                    plsc.addupdate_scatter(hist, [iv], ones)

            start_load(bufs[0], 0)
            last = nblk - 1 if nblk % 2 else nblk - 2

            def blk(bb, _):
                b0 = 2 * bb
                start_load(bufs[1], b0 + 1)
                histogram(bufs[0])
                start_load(bufs[0], jnp.minimum(b0 + 2, last))
                histogram(bufs[1])
                return 0
            lax.fori_loop(0, nblk // 2, blk, 0)
            if nblk % 2:
                histogram(bufs[0])
            else:
                pltpu.make_async_copy(u_hbm.at[pl.ds(0, 1024)], u0, sem0).wait()

            pltpu.sync_copy(hist, out_hbm.at[pl.ds(w * n_acc, n_acc)])

    return k


@functools.lru_cache(maxsize=None)
def _make_scatter_kernel(n_acc, e_pad):
    e_pc = e_pad // NS
    nblk = e_pc // BLK
    assert nblk * BLK == e_pc
    RPB = BLK // 128
    rows_pc = e_pc // 128
    slc = n_acc // NS
    mesh = plsc.VectorSubcoreMesh(core_axis_name="c", subcore_axis_name="s")

    @functools.partial(
        pl.kernel, mesh=mesh,
        compiler_params=pltpu.CompilerParams(use_tc_tiling_on_sc=False, needs_layout_passes=False),
        out_type=jax.ShapeDtypeStruct((4 * n_acc, LN), jnp.float32),
        scratch_types=[
            pltpu.VMEM((RPB, 128), jnp.int32),
            pltpu.VMEM((RPB, 128), jnp.int32),
            pltpu.VMEM((RPB, 128), jnp.int32),
            pltpu.VMEM((BLK, LN), jnp.float32),
            pltpu.VMEM_SHARED((n_acc, LN), jnp.float32),
            pltpu.SemaphoreType.DMA,
            pltpu.SemaphoreType.DMA,
        ],
    )
    def k(y_hbm, u_hbm, v_hbm, out_hbm, u_vm, uo_vm, v_vm, rows_vm, acc, sg, ss):
        c = lax.axis_index("c")
        s = lax.axis_index("s")

        for qi in range(2):
            q = 2 * c + qi
            qoff = q * n_acc

            def zrows(i, _):
                rows_vm[i, :] = jnp.zeros((LN,), jnp.float32)
                return 0
            lax.fori_loop(0, BLK, zrows, 0)
            nz_full = slc // BLK
            for zi in range(nz_full):
                pltpu.sync_copy(rows_vm, acc.at[pl.ds(s * slc + zi * BLK, BLK)])
            rem = slc - nz_full * BLK
            if rem:
                pltpu.sync_copy(rows_vm.at[pl.ds(0, rem)],
                                acc.at[pl.ds(s * slc + nz_full * BLK, rem)])
            plsc.subcore_barrier()

            def blk_body(b, _):
                row0 = s * rows_pc + b * RPB
                pltpu.sync_copy(u_hbm.at[pl.ds(row0, RPB)], u_vm)
                pltpu.sync_copy(v_hbm.at[pl.ds(row0, RPB)], v_vm)
                for j in range(RPB):
                    for l in range(128 // LN):
                        uo_vm[j, pl.ds(l * LN, LN)] = (
                            u_vm[j, pl.ds(l * LN, LN)] + qoff)
                gathers = [
                    pltpu.async_copy(y_hbm.at[uo_vm.at[j]],
                                     rows_vm.at[pl.ds(j * 128, 128)], sg)
                    for j in range(RPB)
                ]
                for g in gathers:
                    g.wait()
                scatters = [
                    pltpu.async_copy(rows_vm.at[pl.ds(j * 128, 128)],
                                     acc.at[v_vm.at[j]], ss, add=True)
                    for j in range(RPB)
                ]
                for sc in scatters:
                    sc.wait()
                return 0
            lax.fori_loop(0, nblk, blk_body, 0)

            plsc.subcore_barrier()
            pltpu.sync_copy(acc.at[pl.ds(s * slc, slc)],
                            out_hbm.at[pl.ds(qoff + s * slc, slc)])
            plsc.subcore_barrier()

    return k


def _quarter(x_pad):
    n_acc = x_pad.shape[0]
    return jnp.transpose(x_pad.reshape(n_acc, 4, LN), (1, 0, 2)).reshape(4 * n_acc, LN)


def _unquarter_rows(tab, idx):
    n_acc = tab.shape[0] // 4
    q = tab.reshape(4, n_acc, LN)[:, idx, :]
    return jnp.transpose(q, (1, 0, 2)).reshape(idx.shape[0], 64)


def _edges(emb_a, edge_index, n):
    na = emb_a.shape[0]
    e = edge_index.shape[1]
    src = edge_index[0]
    dst = edge_index[1] + na
    u = jnp.concatenate([src, dst])
    v = jnp.concatenate([dst, src])
    blk2 = NS * 2 * BLK
    e_pad = ((2 * e + blk2 - 1) // blk2) * blk2
    pad = e_pad - 2 * e
    u = jnp.concatenate([u, jnp.full((pad,), n, jnp.int32)])
    v = jnp.concatenate([v, jnp.full((pad,), n, jnp.int32)])
    return u, v, e_pad


def _propagate_pair(g1, g2, n_acc):
    n = g1[0].shape[0] + g1[1].shape[0]
    u1, v1, e_pad = _edges(g1[0], g1[2], n)
    u2, v2, _ = _edges(g2[0], g2[2], n)

    d1p, d2p = _make_deg_kernel(n_acc, e_pad)(u1, u2)
    scat = _make_scatter_kernel(n_acc, e_pad)

    outs = []
    for (emb_a, emb_b, _), u, v, dp in ((g1, u1, v1, d1p), (g2, u2, v2, d2p)):
        deg = dp.reshape(NC * NS, n_acc).sum(axis=0)
        rs = lax.rsqrt(jnp.maximum(deg, 1.0))[:, None]
        x0 = jnp.zeros((n_acc, 64), jnp.float32).at[:n].set(
            jnp.concatenate([emb_a, emb_b], axis=0))
        rs4 = jnp.tile(rs, (4, 1))
        u2d = u.reshape(e_pad // 128, 128)
        v2d = v.reshape(e_pad // 128, 128)
        y0 = _quarter(x0 * rs)
        t1 = scat(y0, u2d, v2d)
        t2 = scat(t1 * (rs4 * rs4), u2d, v2d)
        outs.append((x0, t1, t2, rs))
    return outs


def _final_rows(x0, t1, t2, rs, idx):
    r = rs[idx]
    return (x0[idx] + r * (_unquarter_rows(t1, idx) + _unquarter_rows(t2, idx))) * (1.0 / 3.0)


def _fuse_rows(e1, e2, Wg, bg):
    h = jnp.concatenate([e1, e2], axis=-1)
    out = jnp.zeros_like(e1)
    for g in range(Wg.shape[0]):
        gate = jax.nn.sigmoid(jnp.dot(h, Wg[g], preferred_element_type=jnp.float32) + bg[g])
        out = out + gate * e1 + (1.0 - gate) * e2
    return out * (1.0 / Wg.shape[0])


def _normalize(z):
    return z / (jnp.sqrt(jnp.sum(z * z, axis=-1, keepdims=True)) + 1e-8)


def _contrast_part(z1_blk, z2_full, base):
    nb = z1_blk.shape[0]
    n = z2_full.shape[0]
    sim = jnp.dot(_normalize(z1_blk), _normalize(z2_full).T,
                  preferred_element_type=jnp.float32) * (1.0 / TEMP)
    m = jnp.max(sim, axis=-1, keepdims=True)
    lse = jnp.log(jnp.sum(jnp.exp(sim - m), axis=-1, keepdims=True)) + m
    rows = jax.lax.broadcasted_iota(jnp.int32, (nb, n), 0)
    cols = jax.lax.broadcasted_iota(jnp.int32, (nb, n), 1)
    diag = jnp.sum(jnp.where(cols == rows + base, sim, 0.0), axis=-1, keepdims=True)
    return jnp.sum(diag - lse)


def _cpt_fuse_kernel_body(c_ref, w_ref, Wg_ref, bg_ref, out_ref):
    out_ref[...] = _fuse_rows(c_ref[...], w_ref[...], Wg_ref[...], bg_ref[...])


def _cpt_fuse(cpt_c, cpt_w, Wg, bg):
    npad = cpt_c.shape[0]
    return pl.pallas_call(
        _cpt_fuse_kernel_body,
        out_shape=jax.ShapeDtypeStruct((npad, 64), jnp.float32),
    )(cpt_c, cpt_w, Wg, bg)


def _head_kernel_body(s_cse, s_wse, s_csc, s_wsc, e_c, e_w, e_w_full,
                      z1c, z2c_full, cpt_mean,
                      Wg_se, bg_se, Wg_sc, bg_sc, Wg_stu, bg_stu, Wg_exer, bg_exer,
                      W1, b1, W2, b2, Wk1, bk1, Wk2, bk2,
                      pred_ref, ks_ref, ce_ref, cc_ref):
    i = pl.program_id(0)
    nb = s_cse.shape[0]
    stu_se = _fuse_rows(s_cse[...], s_wse[...], Wg_se[...], bg_se[...])
    stu_sc = _fuse_rows(s_csc[...], s_wsc[...], Wg_sc[...], bg_sc[...])
    b_stu = _fuse_rows(stu_se, stu_sc, Wg_stu[...], bg_stu[...])
    b_exer = _fuse_rows(e_c[...], e_w[...], Wg_exer[...], bg_exer[...])

    h = jnp.concatenate([b_stu, b_exer, cpt_mean[...]], axis=-1)
    h = jax.nn.relu(jnp.dot(h, W1[...], preferred_element_type=jnp.float32) + b1[...])
    pred_ref[...] = jax.nn.sigmoid(
        jnp.dot(h, W2[...], preferred_element_type=jnp.float32) + b2[...])
    kh = jax.nn.relu(jnp.dot(b_stu, Wk1[...], preferred_element_type=jnp.float32) + bk1[...])
    ks_ref[...] = jax.nn.sigmoid(
        jnp.dot(kh, Wk2[...], preferred_element_type=jnp.float32) + bk2[...])

    base = i * nb
    ce_ref[...] = _contrast_part(e_c[...], e_w_full[...], base).reshape(1, 1, 1)
    cc_ref[...] = _contrast_part(z1c[...], z2c_full[...], base).reshape(1, 1, 1)


def _heads(s_cse, s_wse, s_csc, s_wsc, e_c, e_w, z1c, z2c, cpt_mean,
           Wg_se, bg_se, Wg_sc, bg_sc, Wg_stu, bg_stu, Wg_exer, bg_exer,
           W1, b1, W2, b2, Wk1, bk1, Wk2, bk2):
    B = s_cse.shape[0]
    NB = 512
    G = B // NB
    C_out = Wk2.shape[1]
    b1, b2, bk1, bk2 = b1[None, :], b2[None, :], bk1[None, :], bk2[None, :]

    def rows(i):
        return (i, 0)

    def whole(i):
        return (0, 0)

    row_spec = pl.BlockSpec((NB, 64), rows)
    full_spec = pl.BlockSpec((B, 64), whole)
    weights = (Wg_se, bg_se, Wg_sc, bg_sc, Wg_stu, bg_stu, Wg_exer, bg_exer,
               W1, b1, W2, b2, Wk1, bk1, Wk2, bk2)

    pred, ks, ce, cc = pl.pallas_call(
        _head_kernel_body,
        grid=(G,),
        in_specs=[row_spec] * 6
        + [full_spec]
        + [row_spec, full_spec, row_spec]
        + [pl.BlockSpec(w.shape, lambda i, nd=w.ndim: (0,) * nd)
           for w in weights],
        out_specs=[
            pl.BlockSpec((NB, 1), rows),
            pl.BlockSpec((NB, C_out), rows),
            pl.BlockSpec((1, 1, 1), lambda i: (i, 0, 0)),
            pl.BlockSpec((1, 1, 1), lambda i: (i, 0, 0)),
        ],
        out_shape=[
            jax.ShapeDtypeStruct((B, 1), jnp.float32),
            jax.ShapeDtypeStruct((B, C_out), jnp.float32),
            jax.ShapeDtypeStruct((G, 1, 1), jnp.float32),
            jax.ShapeDtypeStruct((G, 1, 1), jnp.float32),
        ],
    )(s_cse, s_wse, s_csc, s_wsc, e_c, e_w, e_w, z1c, z2c, cpt_mean, *weights)
    c_exer = -jnp.sum(ce) / B
    c_cpt = -jnp.sum(cc) / B
    return pred[:, 0], ks, c_exer, c_cpt


def kernel(stu_ids, exer_ids, cpt_ids, labels, adj_correct_se, adj_wrong_se,
           adj_correct_sc, adj_wrong_sc,
           emb_stu_cse, emb_exer_c, emb_stu_wse, emb_exer_w,
           emb_stu_csc, emb_cpt_c, emb_stu_wsc, emb_cpt_w,
           Wg_se, bg_se, Wg_sc, bg_sc, Wg_stu, bg_stu, Wg_exer, bg_exer,
           Wg_cpt, bg_cpt, W1, b1, W2, b2, Wk1, bk1, Wk2, bk2):
    S = emb_stu_cse.shape[0]
    C = emb_cpt_c.shape[0]
    N_SE = 70144
    N_SC = 51200

    p_cse, p_wse = _propagate_pair(
        (emb_stu_cse, emb_exer_c, adj_correct_se),
        (emb_stu_wse, emb_exer_w, adj_wrong_se), N_SE)
    p_csc, p_wsc = _propagate_pair(
        (emb_stu_csc, emb_cpt_c, adj_correct_sc),
        (emb_stu_wsc, emb_cpt_w, adj_wrong_sc), N_SC)

    exer_nodes = S + exer_ids
    cpt_nodes = S + jnp.arange(C, dtype=jnp.int32)

    stu_c_se_b = _final_rows(*p_cse, stu_ids)
    stu_w_se_b = _final_rows(*p_wse, stu_ids)
    stu_c_sc_b = _final_rows(*p_csc, stu_ids)
    stu_w_sc_b = _final_rows(*p_wsc, stu_ids)
    exer_c_b = _final_rows(*p_cse, exer_nodes)
    exer_w_b = _final_rows(*p_wse, exer_nodes)
    cpt_c_t = _final_rows(*p_csc, cpt_nodes)
    cpt_w_t = _final_rows(*p_wsc, cpt_nodes)

    pad_c = jnp.zeros((1024 - C, 64), jnp.float32)
    cpt_final = _cpt_fuse(jnp.concatenate([cpt_c_t, pad_c]),
                          jnp.concatenate([cpt_w_t, pad_c]), Wg_cpt, bg_cpt)[:C]
    cpt_mean = jnp.mean(cpt_final[cpt_ids], axis=1)

    cpt_batch = cpt_ids[:, 0]
    predictions, knowledge_state, c_exer, c_cpt = _heads(
        stu_c_se_b, stu_w_se_b, stu_c_sc_b, stu_w_sc_b, exer_c_b, exer_w_b,
        cpt_c_t[cpt_batch], cpt_w_t[cpt_batch], cpt_mean,
        Wg_se, bg_se, Wg_sc, bg_sc, Wg_stu, bg_stu, Wg_exer, bg_exer,
        W1, b1, W2, b2, Wk1, bk1, Wk2, bk2)
    return (predictions, knowledge_state, c_exer, c_cpt)


# speedup vs baseline: 1.6538x; 1.2575x over previous
import functools

import jax
import jax.numpy as jnp
from jax import lax
from jax.experimental import pallas as pl
from jax.experimental.pallas import tpu as pltpu
from jax.experimental.pallas import tpu_sc as plsc

TEMP = 0.1
NC = 2
NS = 16
LN = 16

BLK = 2048


@functools.lru_cache(maxsize=None)
def _make_deg_kernel(n_acc, e_pad):
    e_pt = e_pad // (NC * NS)
    nblk = e_pt // 1024
    assert nblk * 1024 == e_pt
    mesh = plsc.VectorSubcoreMesh(core_axis_name="c", subcore_axis_name="s")

    @functools.partial(
        pl.kernel, mesh=mesh,
        compiler_params=pltpu.CompilerParams(use_tc_tiling_on_sc=False, needs_layout_passes=False),
        out_type=(jax.ShapeDtypeStruct((NC * NS * n_acc,), jnp.float32),
                  jax.ShapeDtypeStruct((NC * NS * n_acc,), jnp.float32)),
        scratch_types=[
            pltpu.VMEM((1024,), jnp.int32),
            pltpu.VMEM((1024,), jnp.int32),
            pltpu.VMEM((n_acc,), jnp.float32),
            pltpu.SemaphoreType.DMA,
            pltpu.SemaphoreType.DMA,
        ],
    )
    def k(ua_hbm, ub_hbm, outa_hbm, outb_hbm, u0, u1, hist, sem0, sem1):
        c = lax.axis_index("c")
        s = lax.axis_index("s")
        w = c * NS + s
        ones = jnp.ones((LN,), jnp.float32)
        bufs = ((u0, sem0), (u1, sem1))

        for u_hbm, out_hbm in ((ua_hbm, outa_hbm), (ub_hbm, outb_hbm)):
            def zh(i, _):
                hist[pl.ds(i * LN, LN)] = jnp.zeros((LN,), jnp.float32)
                return 0
            lax.fori_loop(0, n_acc // LN, zh, 0)

            def start_load(buf, b):
                u_vm, sem = buf
                return pltpu.async_copy(
                    u_hbm.at[pl.ds(w * e_pt + b * 1024, 1024)], u_vm, sem)

            def histogram(buf):
                u_vm, sem = buf
                pltpu.make_async_copy(
                    u_hbm.at[pl.ds(0, 1024)], u_vm, sem).wait()
                for j in range(1024 // LN):
                    iv = u_vm[pl.ds(j * LN, LN)]
                    plsc.addupdate_scatter(hist, [iv], ones)

            start_load(bufs[0], 0)
            last = nblk - 1 if nblk % 2 else nblk - 2

            def blk(bb, _):
                b0 = 2 * bb
                start_load(bufs[1], b0 + 1)
                histogram(bufs[0])
                start_load(bufs[0], jnp.minimum(b0 + 2, last))
                histogram(bufs[1])
                return 0
            lax.fori_loop(0, nblk // 2, blk, 0)
            if nblk % 2:
                histogram(bufs[0])
            else:
                pltpu.make_async_copy(u_hbm.at[pl.ds(0, 1024)], u0, sem0).wait()

            pltpu.sync_copy(hist, out_hbm.at[pl.ds(w * n_acc, n_acc)])

    return k


@functools.lru_cache(maxsize=None)
def _make_scatter_kernel(n_acc, e_pad):
    e_pc = e_pad // NS
    nblk = e_pc // BLK
    assert nblk * BLK == e_pc
    RPB = BLK // 128
    rows_pc = e_pc // 128
    slc = n_acc // NS
    mesh = plsc.VectorSubcoreMesh(core_axis_name="c", subcore_axis_name="s")

    @functools.partial(
        pl.kernel, mesh=mesh,
        compiler_params=pltpu.CompilerParams(use_tc_tiling_on_sc=False, needs_layout_passes=False),
        out_type=jax.ShapeDtypeStruct((4 * n_acc, LN), jnp.float32),
        scratch_types=[
            pltpu.VMEM((RPB, 128), jnp.int32),
            pltpu.VMEM((RPB, 128), jnp.int32),
            pltpu.VMEM((RPB, 128), jnp.int32),
            pltpu.VMEM((BLK, LN), jnp.float32),
            pltpu.VMEM_SHARED((n_acc, LN), jnp.float32),
            pltpu.SemaphoreType.DMA,
            pltpu.SemaphoreType.DMA,
        ],
    )
    def k(y_hbm, u_hbm, v_hbm, out_hbm, u_vm, uo_vm, v_vm, rows_vm, acc, sg, ss):
        c = lax.axis_index("c")
        s = lax.axis_index("s")

        for qi in range(2):
            q = 2 * c + qi
            qoff = q * n_acc

            def zrows(i, _):
                rows_vm[i, :] = jnp.zeros((LN,), jnp.float32)
                return 0
            lax.fori_loop(0, BLK, zrows, 0)
            nz_full = slc // BLK
            for zi in range(nz_full):
                pltpu.sync_copy(rows_vm, acc.at[pl.ds(s * slc + zi * BLK, BLK)])
            rem = slc - nz_full * BLK
            if rem:
                pltpu.sync_copy(rows_vm.at[pl.ds(0, rem)],
                                acc.at[pl.ds(s * slc + nz_full * BLK, rem)])
            plsc.subcore_barrier()

            def blk_body(b, _):
                row0 = s * rows_pc + b * RPB
                pltpu.sync_copy(u_hbm.at[pl.ds(row0, RPB)], u_vm)
                pltpu.sync_copy(v_hbm.at[pl.ds(row0, RPB)], v_vm)
                for j in range(RPB):
                    for l in range(128 // LN):
                        uo_vm[j, pl.ds(l * LN, LN)] = (
                            u_vm[j, pl.ds(l * LN, LN)] + qoff)
                gathers = [
                    pltpu.async_copy(y_hbm.at[uo_vm.at[j]],
                                     rows_vm.at[pl.ds(j * 128, 128)], sg)
                    for j in range(RPB)
                ]
                for g in gathers:
                    g.wait()
                scatters = [
                    pltpu.async_copy(rows_vm.at[pl.ds(j * 128, 128)],
                                     acc.at[v_vm.at[j]], ss, add=True)
                    for j in range(RPB)
                ]
                for sc in scatters:
                    sc.wait()
                return 0
            lax.fori_loop(0, nblk, blk_body, 0)

            plsc.subcore_barrier()
            pltpu.sync_copy(acc.at[pl.ds(s * slc, slc)],
                            out_hbm.at[pl.ds(qoff + s * slc, slc)])
            plsc.subcore_barrier()

    return k


def _quarter(x_pad):
    n_acc = x_pad.shape[0]
    return jnp.transpose(x_pad.reshape(n_acc, 4, LN), (1, 0, 2)).reshape(4 * n_acc, LN)


def _unquarter_rows(tab, idx):
    n_acc = tab.shape[0] // 4
    q = tab.reshape(4, n_acc, LN)[:, idx, :]
    return jnp.transpose(q, (1, 0, 2)).reshape(idx.shape[0], 64)


def _edges(emb_a, edge_index, n):
    na = emb_a.shape[0]
    e = edge_index.shape[1]
    src = edge_index[0]
    dst = edge_index[1] + na
    u = jnp.concatenate([src, dst])
    v = jnp.concatenate([dst, src])
    blk2 = NS * BLK
    e_pad = ((2 * e + blk2 - 1) // blk2) * blk2
    pad = e_pad - 2 * e
    dummy = n + (jnp.arange(pad, dtype=jnp.int32) % 128)
    u = jnp.concatenate([u, dummy])
    v = jnp.concatenate([v, dummy])
    return u, v, e_pad


def _propagate_pair(g1, g2, n_acc):
    n = g1[0].shape[0] + g1[1].shape[0]
    u1, v1, e_pad = _edges(g1[0], g1[2], n)
    u2, v2, _ = _edges(g2[0], g2[2], n)

    d1p, d2p = _make_deg_kernel(n_acc, e_pad)(u1, u2)
    scat = _make_scatter_kernel(n_acc, e_pad)

    outs = []
    for (emb_a, emb_b, _), u, v, dp in ((g1, u1, v1, d1p), (g2, u2, v2, d2p)):
        deg = dp.reshape(NC * NS, n_acc).sum(axis=0)
        rs = lax.rsqrt(jnp.maximum(deg, 1.0))[:, None]
        x0 = jnp.zeros((n_acc, 64), jnp.float32).at[:n].set(
            jnp.concatenate([emb_a, emb_b], axis=0))
        rs4 = jnp.tile(rs, (4, 1))
        u2d = u.reshape(e_pad // 128, 128)
        v2d = v.reshape(e_pad // 128, 128)
        y0 = _quarter(x0 * rs)
        t1 = scat(y0, u2d, v2d)
        t2 = scat(t1 * (rs4 * rs4), u2d, v2d)
        outs.append((x0, t1, t2, rs))
    return outs


def _final_rows(x0, t1, t2, rs, idx):
    r = rs[idx]
    return (x0[idx] + r * (_unquarter_rows(t1, idx) + _unquarter_rows(t2, idx))) * (1.0 / 3.0)


def _fuse_rows(e1, e2, Wg, bg):
    h = jnp.concatenate([e1, e2], axis=-1)
    out = jnp.zeros_like(e1)
    for g in range(Wg.shape[0]):
        gate = jax.nn.sigmoid(jnp.dot(h, Wg[g], preferred_element_type=jnp.float32) + bg[g])
        out = out + gate * e1 + (1.0 - gate) * e2
    return out * (1.0 / Wg.shape[0])


def _normalize(z):
    return z / (jnp.sqrt(jnp.sum(z * z, axis=-1, keepdims=True)) + 1e-8)


def _contrast_part(z1_blk, z2_full, base):
    nb = z1_blk.shape[0]
    n = z2_full.shape[0]
    sim = jnp.dot(_normalize(z1_blk), _normalize(z2_full).T,
                  preferred_element_type=jnp.float32) * (1.0 / TEMP)
    m = jnp.max(sim, axis=-1, keepdims=True)
    lse = jnp.log(jnp.sum(jnp.exp(sim - m), axis=-1, keepdims=True)) + m
    rows = jax.lax.broadcasted_iota(jnp.int32, (nb, n), 0)
    cols = jax.lax.broadcasted_iota(jnp.int32, (nb, n), 1)
    diag = jnp.sum(jnp.where(cols == rows + base, sim, 0.0), axis=-1, keepdims=True)
    return jnp.sum(diag - lse)


def _cpt_fuse_kernel_body(c_ref, w_ref, Wg_ref, bg_ref, out_ref):
    out_ref[...] = _fuse_rows(c_ref[...], w_ref[...], Wg_ref[...], bg_ref[...])


def _cpt_fuse(cpt_c, cpt_w, Wg, bg):
    npad = cpt_c.shape[0]
    return pl.pallas_call(
        _cpt_fuse_kernel_body,
        out_shape=jax.ShapeDtypeStruct((npad, 64), jnp.float32),
    )(cpt_c, cpt_w, Wg, bg)


def _head_kernel_body(s_cse, s_wse, s_csc, s_wsc, e_c, e_w, e_w_full,
                      z1c, z2c_full, cpt_mean,
                      Wg_se, bg_se, Wg_sc, bg_sc, Wg_stu, bg_stu, Wg_exer, bg_exer,
                      W1, b1, W2, b2, Wk1, bk1, Wk2, bk2,
                      pred_ref, ks_ref, ce_ref, cc_ref):
    i = pl.program_id(0)
    nb = s_cse.shape[0]
    stu_se = _fuse_rows(s_cse[...], s_wse[...], Wg_se[...], bg_se[...])
    stu_sc = _fuse_rows(s_csc[...], s_wsc[...], Wg_sc[...], bg_sc[...])
    b_stu = _fuse_rows(stu_se, stu_sc, Wg_stu[...], bg_stu[...])
    b_exer = _fuse_rows(e_c[...], e_w[...], Wg_exer[...], bg_exer[...])

    h = jnp.concatenate([b_stu, b_exer, cpt_mean[...]], axis=-1)
    h = jax.nn.relu(jnp.dot(h, W1[...], preferred_element_type=jnp.float32) + b1[...])
    pred_ref[...] = jax.nn.sigmoid(
        jnp.dot(h, W2[...], preferred_element_type=jnp.float32) + b2[...])
    kh = jax.nn.relu(jnp.dot(b_stu, Wk1[...], preferred_element_type=jnp.float32) + bk1[...])
    ks_ref[...] = jax.nn.sigmoid(
        jnp.dot(kh, Wk2[...], preferred_element_type=jnp.float32) + bk2[...])

    base = i * nb
    ce_ref[...] = _contrast_part(e_c[...], e_w_full[...], base).reshape(1, 1, 1)
    cc_ref[...] = _contrast_part(z1c[...], z2c_full[...], base).reshape(1, 1, 1)


def _heads(s_cse, s_wse, s_csc, s_wsc, e_c, e_w, z1c, z2c, cpt_mean,
           Wg_se, bg_se, Wg_sc, bg_sc, Wg_stu, bg_stu, Wg_exer, bg_exer,
           W1, b1, W2, b2, Wk1, bk1, Wk2, bk2):
    B = s_cse.shape[0]
    NB = 512
    G = B // NB
    C_out = Wk2.shape[1]
    b1, b2, bk1, bk2 = b1[None, :], b2[None, :], bk1[None, :], bk2[None, :]

    def rows(i):
        return (i, 0)

    def whole(i):
        return (0, 0)

    row_spec = pl.BlockSpec((NB, 64), rows)
    full_spec = pl.BlockSpec((B, 64), whole)
    weights = (Wg_se, bg_se, Wg_sc, bg_sc, Wg_stu, bg_stu, Wg_exer, bg_exer,
               W1, b1, W2, b2, Wk1, bk1, Wk2, bk2)

    pred, ks, ce, cc = pl.pallas_call(
        _head_kernel_body,
        grid=(G,),
        in_specs=[row_spec] * 6
        + [full_spec]
        + [row_spec, full_spec, row_spec]
        + [pl.BlockSpec(w.shape, lambda i, nd=w.ndim: (0,) * nd)
           for w in weights],
        out_specs=[
            pl.BlockSpec((NB, 1), rows),
            pl.BlockSpec((NB, C_out), rows),
            pl.BlockSpec((1, 1, 1), lambda i: (i, 0, 0)),
            pl.BlockSpec((1, 1, 1), lambda i: (i, 0, 0)),
        ],
        out_shape=[
            jax.ShapeDtypeStruct((B, 1), jnp.float32),
            jax.ShapeDtypeStruct((B, C_out), jnp.float32),
            jax.ShapeDtypeStruct((G, 1, 1), jnp.float32),
            jax.ShapeDtypeStruct((G, 1, 1), jnp.float32),
        ],
    )(s_cse, s_wse, s_csc, s_wsc, e_c, e_w, e_w, z1c, z2c, cpt_mean, *weights)
    c_exer = -jnp.sum(ce) / B
    c_cpt = -jnp.sum(cc) / B
    return pred[:, 0], ks, c_exer, c_cpt


def kernel(stu_ids, exer_ids, cpt_ids, labels, adj_correct_se, adj_wrong_se,
           adj_correct_sc, adj_wrong_sc,
           emb_stu_cse, emb_exer_c, emb_stu_wse, emb_exer_w,
           emb_stu_csc, emb_cpt_c, emb_stu_wsc, emb_cpt_w,
           Wg_se, bg_se, Wg_sc, bg_sc, Wg_stu, bg_stu, Wg_exer, bg_exer,
           Wg_cpt, bg_cpt, W1, b1, W2, b2, Wk1, bk1, Wk2, bk2):
    S = emb_stu_cse.shape[0]
    C = emb_cpt_c.shape[0]
    N_SE = 70144
    N_SC = 51200

    p_cse, p_wse = _propagate_pair(
        (emb_stu_cse, emb_exer_c, adj_correct_se),
        (emb_stu_wse, emb_exer_w, adj_wrong_se), N_SE)
    p_csc, p_wsc = _propagate_pair(
        (emb_stu_csc, emb_cpt_c, adj_correct_sc),
        (emb_stu_wsc, emb_cpt_w, adj_wrong_sc), N_SC)

    exer_nodes = S + exer_ids
    cpt_nodes = S + jnp.arange(C, dtype=jnp.int32)

    stu_c_se_b = _final_rows(*p_cse, stu_ids)
    stu_w_se_b = _final_rows(*p_wse, stu_ids)
    stu_c_sc_b = _final_rows(*p_csc, stu_ids)
    stu_w_sc_b = _final_rows(*p_wsc, stu_ids)
    exer_c_b = _final_rows(*p_cse, exer_nodes)
    exer_w_b = _final_rows(*p_wse, exer_nodes)
    cpt_c_t = _final_rows(*p_csc, cpt_nodes)
    cpt_w_t = _final_rows(*p_wsc, cpt_nodes)

    pad_c = jnp.zeros((1024 - C, 64), jnp.float32)
    cpt_final = _cpt_fuse(jnp.concatenate([cpt_c_t, pad_c]),
                          jnp.concatenate([cpt_w_t, pad_c]), Wg_cpt, bg_cpt)[:C]
    cpt_mean = jnp.mean(cpt_final[cpt_ids], axis=1)

    cpt_batch = cpt_ids[:, 0]
    predictions, knowledge_state, c_exer, c_cpt = _heads(
        stu_c_se_b, stu_w_se_b, stu_c_sc_b, stu_w_sc_b, exer_c_b, exer_w_b,
        cpt_c_t[cpt_batch], cpt_w_t[cpt_batch], cpt_mean,
        Wg_se, bg_se, Wg_sc, bg_sc, Wg_stu, bg_stu, Wg_exer, bg_exer,
        W1, b1, W2, b2, Wk1, bk1, Wk2, bk2)
    return (predictions, knowledge_state, c_exer, c_cpt)
